# baseline (speedup 1.0000x reference)
import functools

import jax
import jax.numpy as jnp
from jax import lax
from jax.experimental import pallas as pl
from jax.experimental.pallas import tpu as pltpu
from jax.experimental.pallas import tpu_sc as plsc

_NUM_CORES = 2
_NUM_SUBCORES = 16
_NW = _NUM_CORES * _NUM_SUBCORES

_LANE = 128
_K = 2
_R = 4


def _build(n_total, dim):
  n_per_w = n_total // _NW
  chunk = _K * _LANE
  n_steps = n_per_w // chunk
  n_outer = n_steps // _R
  assert n_outer * _R == n_steps
  rows_per_w = n_per_w // _LANE
  mesh = plsc.VectorSubcoreMesh(core_axis_name="c", subcore_axis_name="s")

  @functools.partial(
      pl.kernel,
      out_type=jax.ShapeDtypeStruct((n_total, dim), jnp.float32),
      mesh=mesh,
      scratch_types=[
          pltpu.VMEM((_R, _K, _LANE), jnp.int32),
          pltpu.VMEM((_R, _K, _LANE, dim), jnp.float32),
          pltpu.SemaphoreType.DMA((_R,)),
          pltpu.SemaphoreType.DMA((_R,)),
          pltpu.SemaphoreType.DMA((_R,)),
      ],
      compiler_params=pltpu.CompilerParams(use_tc_tiling_on_sc=False),
  )
  def lookup(ids_hbm, table_hbm, out_hbm, idx_v, rows_v, isem, gsem, osem):
    wid = lax.axis_index("s") * _NUM_CORES + lax.axis_index("c")
    base = wid * chunk * n_steps
    idx_base = wid * rows_per_w

    def idx_load(step, slot):
      return pltpu.make_async_copy(
          ids_hbm.at[pl.ds(idx_base + step * _K, _K)], idx_v.at[slot],
          isem.at[slot])

    def gather(slot, j):
      return pltpu.make_async_copy(
          table_hbm.at[idx_v.at[slot].at[j]], rows_v.at[slot].at[j],
          gsem.at[slot])

    def writeout(step, slot, j):
      off = base + step * chunk + j * _LANE
      return pltpu.make_async_copy(
          rows_v.at[slot].at[j], out_hbm.at[pl.ds(off, _LANE)], osem.at[slot])

    for s in range(_R):
      idx_load(s, s).start()
    idx_load(0, 0).wait()
    for j in range(_K):
      gather(0, j).start()

    def body(t, carry):
      for s in range(_R):
        g = t * _R + s
        c = (s + 1) % _R
        for j in range(_K):
          gather(s, j).wait()
        for j in range(_K):
          writeout(g, s, j).start()
        @pl.when(g + _R < n_steps)
        def _():
          idx_load(g + _R, s).start()
        @pl.when(g + 1 < n_steps)
        def _():
          @pl.when(g >= _R - 1)
          def _():
            for j in range(_K):
              writeout(g - (_R - 1), c, j).wait()
          idx_load(g + 1, c).wait()
          for j in range(_K):
            gather(c, j).start()
      return carry

    lax.fori_loop(0, n_outer, body, 0)

    for s in range(_R):
      g = n_steps - _R + s
      for j in range(_K):
        writeout(g, g % _R, j).wait()

  return lookup


def kernel(token_ids, W):
  b, s = token_ids.shape
  _, dim = W.shape
  n_total = b * s
  ids = token_ids.reshape(n_total // _LANE, _LANE).astype(jnp.int32)
  out = _build(n_total, dim)(ids, W)
  return out.reshape(b, s, dim)


# speedup vs baseline: 1.0298x; 1.0298x over previous
import functools

import jax
import jax.numpy as jnp
from jax import lax
from jax.experimental import pallas as pl
from jax.experimental.pallas import tpu as pltpu
from jax.experimental.pallas import tpu_sc as plsc

_NUM_CORES = 2
_NUM_SUBCORES = 16
_NW = _NUM_CORES * _NUM_SUBCORES

_SUB = 16
_LANE = 128


def _build(batch, seq, dim):
  b_per_w = batch // _NW
  n_per_w = b_per_w * seq
  n_grp = (seq + _SUB - 1) // _SUB
  pad = n_grp * _SUB
  g_rows = 224
  mesh = plsc.VectorSubcoreMesh(core_axis_name="c", subcore_axis_name="s")

  @functools.partial(
      pl.kernel,
      out_type=jax.ShapeDtypeStruct((batch, seq, dim), jnp.float32),
      mesh=mesh,
      scratch_types=[
          pltpu.VMEM((n_per_w + 2 * _SUB,), jnp.int32),
          pltpu.VMEM((2, 2, _LANE), jnp.int32),
          pltpu.VMEM((2, g_rows, _LANE), jnp.float32),
          pltpu.VMEM((pad, dim), jnp.float32),
          pltpu.SemaphoreType.DMA,
          pltpu.SemaphoreType.DMA((2,)),
          pltpu.SemaphoreType.DMA,
      ],
      compiler_params=pltpu.CompilerParams(use_tc_tiling_on_sc=True),
  )
  def lookup(ids_hbm, pairs_hbm, out_hbm, idx_v, q_v, g_v, o_v,
             isem, gsem, osem):
    wid = lax.axis_index("s") * _NUM_CORES + lax.axis_index("c")
    base = pl.multiple_of(wid * n_per_w, _LANE)
    b0 = wid * b_per_w

    zeros16 = jnp.zeros((_SUB,), jnp.int32)
    idx_v[pl.ds(n_per_w, _SUB)] = zeros16
    idx_v[pl.ds(n_per_w + _SUB, _SUB)] = zeros16
    pltpu.async_copy(ids_hbm.at[pl.ds(base, n_per_w)],
                     idx_v.at[pl.ds(0, n_per_w)], isem)
    pltpu.make_async_copy(ids_hbm.at[pl.ds(base, n_per_w)],
                          idx_v.at[pl.ds(0, n_per_w)], isem).wait()

    def compute_q(k, s):
      for c in range((_LANE + 96) // _SUB):
        ids16 = idx_v[pl.ds(k * seq + c * _SUB, _SUB)]
        q_v[s, c // (_LANE // _SUB), pl.ds((c % (_LANE // _SUB)) * _SUB,
                                           _SUB)] = (
            lax.shift_right_logical(ids16, 1))

    def fire_gather(k, s):
      compute_q(k, s)
      pltpu.async_copy(pairs_hbm.at[q_v.at[s].at[0]],
                       g_v.at[s].at[pl.ds(0, _LANE)], gsem.at[s])
      pltpu.async_copy(pairs_hbm.at[q_v.at[s].at[1].at[pl.ds(0, 96)]],
                       g_v.at[s].at[pl.ds(_LANE, 96)], gsem.at[s])

    def wait_gather(s):
      pltpu.make_async_copy(
          pairs_hbm.at[q_v.at[s].at[0]],
          g_v.at[s].at[pl.ds(0, _LANE)], gsem.at[s]).wait()
      pltpu.make_async_copy(
          pairs_hbm.at[q_v.at[s].at[1].at[pl.ds(0, 96)]],
          g_v.at[s].at[pl.ds(_LANE, 96)], gsem.at[s]).wait()

    def writeout(k):
      return pltpu.make_async_copy(
          o_v.at[pl.ds(0, seq)], out_hbm.at[b0 + k], osem)

    def extract(k, s):
      def group(g, carry):
        ids16 = idx_v[pl.ds(k * seq + g * _SUB, _SUB)]
        par16 = ids16 & 1
        parf16 = par16.astype(jnp.float32)
        for l in range(_SUB):
          spl = jnp.take(parf16, jnp.full((_SUB,), l, jnp.int32))
          cospl = 1.0 - spl
          row = g * _SUB + l
          for c in range(dim // _SUB):
            lo = g_v[s, row, pl.ds(c * _SUB, _SUB)]
            hi = g_v[s, row, pl.ds(dim + c * _SUB, _SUB)]
            o_v[row, pl.ds(c * _SUB, _SUB)] = lo * cospl + hi * spl
        return carry

      lax.fori_loop(0, n_grp, group, 0)

    fire_gather(0, 0)

    def body(t, carry):
      for j in range(2):
        k = t * 2 + j
        s = j
        @pl.when(k + 1 < b_per_w)
        def _():
          fire_gather(k + 1, 1 - s)
        wait_gather(s)
        @pl.when(k >= 1)
        def _():
          writeout(k - 1).wait()
        extract(k, s)
        writeout(k).start()
      return carry

    lax.fori_loop(0, b_per_w // 2, body, 0)

    writeout(b_per_w - 1).wait()

  return lookup


def kernel(token_ids, W):
  b, s = token_ids.shape
  vocab, dim = W.shape
  ids = token_ids.reshape(b * s).astype(jnp.int32)
  pairs = W.reshape(vocab // 2, 2 * dim)
  return _build(b, s, dim)(ids, pairs)
